# Initial kernel scaffold; baseline (speedup 1.0000x reference)
import jax
import jax.numpy as jnp
from jax.experimental import pallas as pl


def kernel(x_s, edge_attr_s, x_t, edge_attr_t, params, edge_index_s, edge_index_t, xs_batch, xt_batch):
    raise NotImplementedError("write your pallas kernel here")


# speedup vs baseline: 7.1983x; 7.1983x over previous
import functools

import jax
import jax.numpy as jnp
import numpy as np
from jax import lax
from jax.experimental import pallas as pl
from jax.experimental.pallas import tpu as pltpu
from jax.experimental.pallas import tpu_sc as plsc

N = 10000
E = 160000
D = 128
G = 64
Np = 10240
Ep = 163840
H16 = 16
CH = 128
NW = 32

SDS = jax.ShapeDtypeStruct
f32 = jnp.float32


def _mesh():
    return plsc.VectorSubcoreMesh(core_axis_name="c", subcore_axis_name="s")


@functools.partial(jax.jit, static_argnames=("din", "W"))
def _lin2(x, Wl, bl, Wr, br, *, din, W):
    BN = 512

    def body(x_ref, wl_ref, bl_ref, wr_ref, br_ref, xl_ref, xr_ref):
        xb = x_ref[...]
        xl_ref[...] = jnp.dot(xb, wl_ref[...], preferred_element_type=f32) + bl_ref[...]
        xr_ref[...] = jnp.dot(xb, wr_ref[...], preferred_element_type=f32) + br_ref[...]

    return pl.pallas_call(
        body,
        grid=(Np // BN,),
        in_specs=[
            pl.BlockSpec((BN, din), lambda i: (i, 0)),
            pl.BlockSpec((din, W), lambda i: (0, 0)),
            pl.BlockSpec((1, W), lambda i: (0, 0)),
            pl.BlockSpec((din, W), lambda i: (0, 0)),
            pl.BlockSpec((1, W), lambda i: (0, 0)),
        ],
        out_specs=[pl.BlockSpec((BN, W), lambda i: (i, 0)),
                   pl.BlockSpec((BN, W), lambda i: (i, 0))],
        out_shape=[SDS((Np, W), f32), SDS((Np, W), f32)],
    )(x, Wl, bl, Wr, br)


@functools.partial(jax.jit, static_argnames=("W",))
def _sc_gather(xl, xr, src, dst, *, W):
    nper = Ep // NW
    nch = nper // CH

    @functools.partial(
        pl.kernel,
        mesh=_mesh(),
        out_type=(SDS((Ep, W), f32), SDS((Ep, W), f32)),
        scratch_types=(pltpu.VMEM((CH,), jnp.int32),
                       pltpu.VMEM((CH, W), f32),
                       pltpu.SemaphoreType.DMA),
    )
    def gat(xl_h, xr_h, src_h, dst_h, xls_o, xrd_o, idx_v, rows_v, sem):
        wid = lax.axis_index("s") * 2 + lax.axis_index("c")
        base0 = wid * nper

        def body(i, carry):
            base = base0 + i * CH
            pltpu.sync_copy(src_h.at[pl.ds(base, CH)], idx_v)
            pltpu.async_copy(xl_h.at[idx_v], rows_v, sem).wait()
            pltpu.sync_copy(rows_v, xls_o.at[pl.ds(base, CH)])
            pltpu.sync_copy(dst_h.at[pl.ds(base, CH)], idx_v)
            pltpu.async_copy(xr_h.at[idx_v], rows_v, sem).wait()
            pltpu.sync_copy(rows_v, xrd_o.at[pl.ds(base, CH)])
            return carry

        lax.fori_loop(0, nch, body, 0)

    return gat(xl, xr, src, dst)


@functools.partial(jax.jit, static_argnames=("W", "ng_s", "fold"))
def _att(xls, xrd, ea, We, attb, S, St, hmask, P1, P2, *, W, ng_s, fold):
    BE = 512
    ng_out = ng_s if fold else ng_s + 1

    def body(xls_ref, xrd_ref, ea_ref, we_ref, attb_ref, s_ref, st_ref, hm_ref,
             p1_ref, p2_ref, *outs):
        pid = pl.program_id(0)
        xlsb = xls_ref[...]
        m = xlsb + xrd_ref[...] + jnp.dot(ea_ref[...], we_ref[...],
                                          preferred_element_type=f32)
        m = jnp.maximum(m, 0.0) + 0.2 * jnp.minimum(m, 0.0)
        logits = jnp.dot(m * attb_ref[...], s_ref[...], preferred_element_type=f32, precision=lax.Precision.HIGHEST)
        ids = lax.broadcasted_iota(jnp.int32, (BE, 1), 0) + pid * BE
        emask = (ids < E).astype(f32)
        ex = jnp.exp(logits) * hm_ref[...] * emask
        scaled = xlsb * jnp.dot(ex, st_ref[...], preferred_element_type=f32, precision=lax.Precision.HIGHEST)
        exg = jnp.dot(ex, p2_ref[...], preferred_element_type=f32, precision=lax.Precision.HIGHEST)
        if fold:
            outs[0][...] = jnp.dot(scaled, p1_ref[...], preferred_element_type=f32, precision=lax.Precision.HIGHEST) + exg
        else:
            for g in range(ng_s):
                outs[g][...] = scaled[:, g * 128:(g + 1) * 128]
            outs[ng_s][...] = exg

    return pl.pallas_call(
        body,
        grid=(Ep // BE,),
        in_specs=[
            pl.BlockSpec((BE, W), lambda i: (i, 0)),
            pl.BlockSpec((BE, W), lambda i: (i, 0)),
            pl.BlockSpec((BE, H16), lambda i: (i, 0)),
            pl.BlockSpec((H16, W), lambda i: (0, 0)),
            pl.BlockSpec((1, W), lambda i: (0, 0)),
            pl.BlockSpec((W, H16), lambda i: (0, 0)),
            pl.BlockSpec((H16, W), lambda i: (0, 0)),
            pl.BlockSpec((1, H16), lambda i: (0, 0)),
            pl.BlockSpec((W, 128), lambda i: (0, 0)),
            pl.BlockSpec((H16, 128), lambda i: (0, 0)),
        ],
        out_specs=[pl.BlockSpec((BE, 128), lambda i: (i, 0)) for _ in range(ng_out)],
        out_shape=[SDS((Ep, 128), f32) for _ in range(ng_out)],
    )(xls, xrd, ea, We, attb, S, St, hmask, P1, P2)


@functools.partial(jax.jit, static_argnames=("ng",))
def _sc_scatter(groups, dst, *, ng):
    nper = Ep // NW
    nch = nper // CH
    rpt = Np // 16
    nz = rpt // CH

    @functools.partial(
        pl.kernel,
        mesh=_mesh(),
        out_type=tuple(SDS((2, Np, 128), f32) for _ in range(ng)),
        scratch_types=(pltpu.VMEM((CH,), jnp.int32),
                       pltpu.VMEM((CH, 128), f32),
                       pltpu.VMEM_SHARED((Np, 128), f32),
                       pltpu.SemaphoreType.DMA),
    )
    def scat(*refs):
        g_refs = refs[0:ng]
        dst_h = refs[ng]
        out_refs = refs[ng + 1:2 * ng + 1]
        idx_v, rows_v, acc_sh, sem = refs[2 * ng + 1:]

        c = lax.axis_index("c")
        s = lax.axis_index("s")
        wid = s * 2 + c
        base0 = wid * nper
        zero16 = jnp.zeros((16,), f32)

        for g in range(ng):
            def zrow(i, carry):
                for j in range(8):
                    rows_v[i, pl.ds(j * 16, 16)] = zero16
                return carry
            lax.fori_loop(0, CH, zrow, 0)
            for z in range(nz):
                pltpu.sync_copy(rows_v, acc_sh.at[pl.ds(s * rpt + z * CH, CH)])
            plsc.subcore_barrier()

            def body(i, carry):
                base = base0 + i * CH
                pltpu.sync_copy(dst_h.at[pl.ds(base, CH)], idx_v)
                pltpu.sync_copy(g_refs[g].at[pl.ds(base, CH)], rows_v)
                pltpu.sync_copy(rows_v, acc_sh.at[idx_v], add=True)
                return carry

            lax.fori_loop(0, nch, body, 0)
            plsc.subcore_barrier()

            for z in range(nz):
                rb = s * rpt + z * CH
                pltpu.sync_copy(acc_sh.at[pl.ds(rb, CH)], out_refs[g].at[c, pl.ds(rb, CH)])

    return scat(*groups, dst)


@functools.partial(jax.jit, static_argnames=("W", "ng_s"))
def _acc(parts, den, bias, Sd, Pn, *, W, ng_s):
    BA = 512

    def body(*refs):
        p_refs = refs[0:ng_s]
        den_ref, bias_ref, sd_ref, pn_ref, y_ref, stats_ref, acc = refs[ng_s:]
        pid = pl.program_id(0)

        @pl.when(pid == 0)
        def _():
            acc[...] = jnp.zeros_like(acc)

        if ng_s == 1:
            p = jnp.dot(p_refs[0][0] + p_refs[0][1], pn_ref[0],
                        preferred_element_type=f32,
                        precision=lax.Precision.HIGHEST)
        else:
            p = jnp.concatenate([r[0] + r[1] for r in p_refs], axis=1)
        dsum = den_ref[0] + den_ref[1]
        dexp = jnp.dot(dsum, sd_ref[...], preferred_element_type=f32, precision=lax.Precision.HIGHEST)
        y = p / (dexp + 1e-16) + bias_ref[...]
        y_ref[...] = y
        ids = lax.broadcasted_iota(jnp.int32, (BA, 1), 0) + pid * BA
        rmask = (ids < N).astype(f32)
        ym = y * rmask
        acc[0:1, :] += jnp.sum(ym, axis=0, keepdims=True)
        acc[1:2, :] += jnp.sum(ym * y, axis=0, keepdims=True)

        @pl.when(pid == Np // BA - 1)
        def _():
            stats_ref[...] = acc[...]

    return pl.pallas_call(
        body,
        grid=(Np // BA,),
        in_specs=[pl.BlockSpec((2, BA, 128), lambda i: (0, i, 0)) for _ in range(ng_s)]
                 + [pl.BlockSpec((2, BA, 128), lambda i: (0, i, 0)),
                    pl.BlockSpec((1, W), lambda i: (0, 0)),
                    pl.BlockSpec((128, W), lambda i: (0, 0)),
                    pl.BlockSpec((1, 128, W), lambda i: (0, 0, 0))],
        out_specs=[pl.BlockSpec((BA, W), lambda i: (i, 0)),
                   pl.BlockSpec((8, W), lambda i: (0, 0))],
        out_shape=[SDS((Np, W), f32), SDS((8, W), f32)],
        scratch_shapes=[pltpu.VMEM((8, W), f32)],
    )(*parts, den, bias, Sd, Pn)


@jax.jit
def _norm(y, stats, ms, w, b):
    BN = 1024
    W = y.shape[1]

    def body(y_ref, st_ref, ms_ref, w_ref, b_ref, o_ref):
        s = st_ref[0:1, :] * (1.0 / N)
        sq = st_ref[1:2, :] * (1.0 / N)
        mm = s * ms_ref[...]
        var = sq - 2.0 * mm * s + mm * mm
        inv = lax.rsqrt(var + 1e-5)
        r = (y_ref[...] - mm) * inv * w_ref[...] + b_ref[...]
        o_ref[...] = jnp.maximum(r, 0.0) + 0.01 * jnp.minimum(r, 0.0)

    return pl.pallas_call(
        body,
        grid=(Np // BN,),
        in_specs=[pl.BlockSpec((BN, W), lambda i: (i, 0)),
                  pl.BlockSpec((8, W), lambda i: (0, 0)),
                  pl.BlockSpec((1, W), lambda i: (0, 0)),
                  pl.BlockSpec((1, W), lambda i: (0, 0)),
                  pl.BlockSpec((1, W), lambda i: (0, 0))],
        out_specs=pl.BlockSpec((BN, W), lambda i: (i, 0)),
        out_shape=SDS((Np, W), f32),
    )(y, stats, ms, w, b)


@jax.jit
def _gmp(x3, batch3d):
    BB = 512
    NB = Np // BB

    def body(x_ref, b_ref, xs_ref, accv, accc):
        pid = pl.program_id(0)

        @pl.when(pid == 0)
        def _():
            accv[...] = jnp.zeros_like(accv)
            accc[...] = jnp.zeros_like(accc)

        bvals = b_ref[0]
        oh = jnp.equal(lax.broadcasted_iota(jnp.int32, (G, BB), 0), bvals).astype(f32)
        accv[...] += jnp.dot(oh, x_ref[...], preferred_element_type=f32, precision=lax.Precision.HIGHEST)
        accc[...] += jnp.dot(oh, jnp.ones((BB, 16), f32), preferred_element_type=f32, precision=lax.Precision.HIGHEST)

        @pl.when(pid == NB - 1)
        def _():
            xs_ref[...] = accv[...] / jnp.maximum(accc[...], 1.0)

    return pl.pallas_call(
        body,
        grid=(NB,),
        in_specs=[pl.BlockSpec((BB, 16), lambda i: (i, 0)),
                  pl.BlockSpec((1, 1, BB), lambda i: (i, 0, 0))],
        out_specs=pl.BlockSpec((G, 16), lambda i: (0, 0)),
        out_shape=SDS((G, 16), f32),
        scratch_shapes=[pltpu.VMEM((G, 16), f32), pltpu.VMEM((G, 16), f32)],
    )(x3, batch3d)


@jax.jit
def _mlp(xs, W1, b1, w1n, b1n, W2, b2, w2n, b2n, W3, b3):
    def body(xs_ref, W1r, b1r, w1nr, b1nr, W2r, b2r, w2nr, b2nr, W3r, b3r,
             out_ref, sig_ref):
        def bn(h, wv, bv):
            m = jnp.mean(h, axis=0, keepdims=True)
            v = jnp.mean((h - m) * (h - m), axis=0, keepdims=True)
            return (h - m) * lax.rsqrt(v + 1e-5) * wv + bv

        lr = lambda z: jnp.maximum(z, 0.0) + 0.01 * jnp.minimum(z, 0.0)
        h = lr(bn(jnp.dot(xs_ref[...], W1r[...], preferred_element_type=f32)
                  + b1r[...], w1nr[...], b1nr[...]))
        h = lr(bn(jnp.dot(h, W2r[...], preferred_element_type=f32)
                  + b2r[...], w2nr[...], b2nr[...]))
        out = jnp.dot(h, W3r[...], preferred_element_type=f32) + b3r[...]
        out_ref[...] = out
        sig_ref[...] = jax.nn.sigmoid(out)

    return pl.pallas_call(
        body,
        out_shape=[SDS((G, 1408), f32), SDS((G, 1408), f32)],
    )(xs, W1, b1, w1n, b1n, W2, b2, w2n, b2n, W3, b3)


def _layer(x, src, dst, ea, p, gp, Hh, C):
    W = Hh * C
    fold = (W < 128)
    Wp = max(W, 128)
    ng_s = Wp // 128 if not fold else 1
    din = x.shape[1]

    S = np.zeros((Wp, H16), np.float32)
    for h in range(Hh):
        S[h * C:(h + 1) * C, h] = 1.0
    hmask = np.zeros((1, H16), np.float32)
    hmask[0, :Hh] = 1.0
    P1 = np.zeros((Wp, 128), np.float32)
    P2 = np.zeros((H16, 128), np.float32)
    if fold:
        P1[:W, :W] = np.eye(W, dtype=np.float32)
        for h in range(Hh):
            P2[h, W + h] = 1.0
        Sd = np.zeros((128, W), np.float32)
        for h in range(Hh):
            Sd[W + h, h * C:(h + 1) * C] = 1.0
    else:
        P2[:H16, :H16] = np.eye(H16, dtype=np.float32)
        Sd = np.zeros((128, W), np.float32)
        for h in range(Hh):
            Sd[h, h * C:(h + 1) * C] = 1.0
    Pn = np.zeros((1, 128, W), np.float32)
    Pn[0, :min(128, W), :min(128, W)] = np.eye(min(128, W), dtype=np.float32)

    padW = ((0, 0), (0, Wp - W))
    Wl = jnp.pad(p['Wl'], padW)
    Wr = jnp.pad(p['Wr'], padW)
    bl = jnp.pad(p['bl'], (0, Wp - W)).reshape(1, Wp)
    br = jnp.pad(p['br'], (0, Wp - W)).reshape(1, Wp)
    attb = jnp.pad(p['att'].reshape(1, W), ((0, 0), (0, Wp - W)))
    We = jnp.pad(p['We'], ((0, H16 - p['We'].shape[0]), (0, Wp - W)))

    xl, xr = _lin2(x, Wl, bl, Wr, br, din=din, W=Wp)
    xls, xrd = _sc_gather(xl, xr, src, dst, W=Wp)
    groups = _att(xls, xrd, ea, We, attb, jnp.asarray(S), jnp.asarray(S.T),
                  jnp.asarray(hmask), jnp.asarray(P1), jnp.asarray(P2),
                  W=Wp, ng_s=ng_s, fold=fold)
    gparts = _sc_scatter(list(groups), dst, ng=len(groups))
    sparts = list(gparts[:ng_s]) if not fold else [gparts[0]]
    dpart = gparts[-1]
    y, stats = _acc(sparts, dpart, p['bias'].reshape(1, W), jnp.asarray(Sd),
                    jnp.asarray(Pn), W=W, ng_s=ng_s)
    return _norm(y, stats, gp['ms'].reshape(1, W), gp['w'].reshape(1, W),
                 gp['b'].reshape(1, W))


def kernel(x_s, edge_attr_s, x_t, edge_attr_t, params, edge_index_s,
           edge_index_t, xs_batch, xt_batch):
    p = params
    x = jnp.pad(x_s, ((0, Np - N), (0, 0)))
    src = jnp.concatenate([edge_index_s[0], jnp.full((Ep - E,), N, jnp.int32)])
    dst = jnp.concatenate([edge_index_s[1], jnp.full((Ep - E,), N, jnp.int32)])
    ea = jnp.pad(edge_attr_s, ((0, Ep - E), (0, H16 - 9)))
    batch3d = jnp.concatenate([xs_batch, jnp.full((Np - N,), G, jnp.int32)]
                              ).reshape(Np // 512, 1, 512)

    x1 = _layer(x, src, dst, ea, p['g1'], p['gn1'], 8, 64)
    x2 = _layer(x1, src, dst, ea, p['g2'], p['gn2'], 4, 32)
    x3 = _layer(x2, src, dst, ea, p['g3'], p['gn3'], 1, 16)
    xs = _gmp(x3, batch3d)

    def pad_lin(lp, ko, no):
        return (jnp.pad(lp['W'], ((0, ko - lp['W'].shape[0]), (0, no - lp['W'].shape[1]))),
                jnp.pad(lp['b'], (0, no - lp['b'].shape[0])).reshape(1, no))

    W1, b1 = pad_lin(p['lin1'], 16, 384)
    W2, b2 = pad_lin(p['lin2'], 384, 768)
    W3, b3 = pad_lin(p['lin3'], 768, 1408)
    w1n = jnp.pad(p['bn1']['w'], (0, 384 - 329)).reshape(1, 384)
    b1n = jnp.pad(p['bn1']['b'], (0, 384 - 329)).reshape(1, 384)
    w2n = jnp.pad(p['bn2']['w'], (0, 768 - 658)).reshape(1, 768)
    b2n = jnp.pad(p['bn2']['b'], (0, 768 - 658)).reshape(1, 768)
    out_p, sig_p = _mlp(xs, W1, b1, w1n, b1n, W2, b2, w2n, b2n, W3, b3)
    return (out_p[:, :1317], sig_p[:, :1317])


# speedup vs baseline: 8.6170x; 1.1971x over previous
import functools

import jax
import jax.numpy as jnp
import numpy as np
from jax import lax
from jax.experimental import pallas as pl
from jax.experimental.pallas import tpu as pltpu
from jax.experimental.pallas import tpu_sc as plsc

N = 10000
E = 160000
D = 128
G = 64
Np = 10240
Ep = 163840
H16 = 16
CH = 128
NW = 32

SDS = jax.ShapeDtypeStruct
f32 = jnp.float32


def _mesh():
    return plsc.VectorSubcoreMesh(core_axis_name="c", subcore_axis_name="s")


@functools.partial(jax.jit, static_argnames=("din", "W"))
def _lin2(x, Wl, bl, Wr, br, *, din, W):
    BN = 512

    def body(x_ref, wl_ref, bl_ref, wr_ref, br_ref, xl_ref, xr_ref):
        xb = x_ref[...]
        xl_ref[...] = jnp.dot(xb, wl_ref[...], preferred_element_type=f32) + bl_ref[...]
        xr_ref[...] = jnp.dot(xb, wr_ref[...], preferred_element_type=f32) + br_ref[...]

    return pl.pallas_call(
        body,
        grid=(Np // BN,),
        in_specs=[
            pl.BlockSpec((BN, din), lambda i: (i, 0)),
            pl.BlockSpec((din, W), lambda i: (0, 0)),
            pl.BlockSpec((1, W), lambda i: (0, 0)),
            pl.BlockSpec((din, W), lambda i: (0, 0)),
            pl.BlockSpec((1, W), lambda i: (0, 0)),
        ],
        out_specs=[pl.BlockSpec((BN, W), lambda i: (i, 0)),
                   pl.BlockSpec((BN, W), lambda i: (i, 0))],
        out_shape=[SDS((Np, W), f32), SDS((Np, W), f32)],
    )(x, Wl, bl, Wr, br)


@functools.partial(jax.jit, static_argnames=("W",))
def _sc_gather(xl, xr, src, dst, *, W):
    ch = 80 if W > 128 else CH
    nper = Ep // NW
    nch = nper // ch

    @functools.partial(
        pl.kernel,
        mesh=_mesh(),
        out_type=(SDS((Ep, W), f32), SDS((Ep, W), f32)),
        scratch_types=(pltpu.VMEM((ch,), jnp.int32),
                       pltpu.VMEM((ch,), jnp.int32),
                       pltpu.VMEM((ch, W), f32),
                       pltpu.VMEM((ch, W), f32),
                       pltpu.SemaphoreType.DMA,
                       pltpu.SemaphoreType.DMA),
    )
    def gat(xl_h, xr_h, src_h, dst_h, xls_o, xrd_o,
            sidx, didx, rows0, rows1, sem0, sem1):
        wid = lax.axis_index("s") * 2 + lax.axis_index("c")
        base0 = wid * nper

        def body(i, carry):
            base = base0 + i * ch
            pltpu.sync_copy(src_h.at[pl.ds(base, ch)], sidx)
            pltpu.sync_copy(dst_h.at[pl.ds(base, ch)], didx)
            d0 = pltpu.async_copy(xl_h.at[sidx], rows0, sem0)
            d1 = pltpu.async_copy(xr_h.at[didx], rows1, sem1)
            d0.wait()
            pltpu.sync_copy(rows0, xls_o.at[pl.ds(base, ch)])
            d1.wait()
            pltpu.sync_copy(rows1, xrd_o.at[pl.ds(base, ch)])
            return carry

        lax.fori_loop(0, nch, body, 0)

    return gat(xl, xr, src, dst)


@functools.partial(jax.jit, static_argnames=("W", "ng_s", "fold"))
def _att(xls, xrd, ea, We, attb, S, St, hmask, P1, P2, *, W, ng_s, fold):
    BE = 512
    ng_out = ng_s if fold else ng_s + 1

    def body(xls_ref, xrd_ref, ea_ref, we_ref, attb_ref, s_ref, st_ref, hm_ref,
             p1_ref, p2_ref, *outs):
        pid = pl.program_id(0)
        xlsb = xls_ref[...]
        m = xlsb + xrd_ref[...] + jnp.dot(ea_ref[...], we_ref[...],
                                          preferred_element_type=f32)
        m = jnp.maximum(m, 0.0) + 0.2 * jnp.minimum(m, 0.0)
        logits = jnp.dot(m * attb_ref[...], s_ref[...], preferred_element_type=f32, precision=lax.Precision.HIGHEST)
        ids = lax.broadcasted_iota(jnp.int32, (BE, 1), 0) + pid * BE
        emask = (ids < E).astype(f32)
        ex = jnp.exp(logits) * hm_ref[...] * emask
        scaled = xlsb * jnp.dot(ex, st_ref[...], preferred_element_type=f32, precision=lax.Precision.HIGHEST)
        exg = jnp.dot(ex, p2_ref[...], preferred_element_type=f32, precision=lax.Precision.HIGHEST)
        if fold:
            outs[0][...] = jnp.dot(scaled, p1_ref[...], preferred_element_type=f32, precision=lax.Precision.HIGHEST) + exg
        else:
            for g in range(ng_s):
                outs[g][...] = scaled[:, g * 128:(g + 1) * 128]
            outs[ng_s][...] = exg

    return pl.pallas_call(
        body,
        grid=(Ep // BE,),
        in_specs=[
            pl.BlockSpec((BE, W), lambda i: (i, 0)),
            pl.BlockSpec((BE, W), lambda i: (i, 0)),
            pl.BlockSpec((BE, H16), lambda i: (i, 0)),
            pl.BlockSpec((H16, W), lambda i: (0, 0)),
            pl.BlockSpec((1, W), lambda i: (0, 0)),
            pl.BlockSpec((W, H16), lambda i: (0, 0)),
            pl.BlockSpec((H16, W), lambda i: (0, 0)),
            pl.BlockSpec((1, H16), lambda i: (0, 0)),
            pl.BlockSpec((W, 128), lambda i: (0, 0)),
            pl.BlockSpec((H16, 128), lambda i: (0, 0)),
        ],
        out_specs=[pl.BlockSpec((BE, 128), lambda i: (i, 0)) for _ in range(ng_out)],
        out_shape=[SDS((Ep, 128), f32) for _ in range(ng_out)],
    )(xls, xrd, ea, We, attb, S, St, hmask, P1, P2)


@functools.partial(jax.jit, static_argnames=("ng",))
def _sc_scatter(groups, dst, *, ng):
    nper = Ep // NW
    nch = nper // CH
    rpt = Np // 16
    nz = rpt // CH

    @functools.partial(
        pl.kernel,
        mesh=_mesh(),
        out_type=tuple(SDS((2, Np, 128), f32) for _ in range(ng)),
        scratch_types=(pltpu.VMEM((CH,), jnp.int32),
                       pltpu.VMEM((CH,), jnp.int32),
                       pltpu.VMEM((CH, 128), f32),
                       pltpu.VMEM((CH, 128), f32),
                       pltpu.VMEM_SHARED((Np, 128), f32),
                       pltpu.SemaphoreType.DMA,
                       pltpu.SemaphoreType.DMA,
                       pltpu.SemaphoreType.DMA,
                       pltpu.SemaphoreType.DMA),
    )
    def scat(*refs):
        g_refs = refs[0:ng]
        dst_h = refs[ng]
        out_refs = refs[ng + 1:2 * ng + 1]
        idx0, idx1, rows0, rows1, acc_sh, lsem0, lsem1, ssem0, ssem1 = refs[2 * ng + 1:]
        idx_b = (idx0, idx1)
        rows_b = (rows0, rows1)
        lsem_b = (lsem0, lsem1)
        ssem_b = (ssem0, ssem1)
        rows_v = rows0

        c = lax.axis_index("c")
        s = lax.axis_index("s")
        wid = s * 2 + c
        base0 = wid * nper
        zero16 = jnp.zeros((16,), f32)

        for g in range(ng):
            def zrow(i, carry):
                for j in range(8):
                    rows_v[i, pl.ds(j * 16, 16)] = zero16
                return carry
            lax.fori_loop(0, CH, zrow, 0)
            for z in range(nz):
                pltpu.sync_copy(rows_v, acc_sh.at[pl.ds(s * rpt + z * CH, CH)])
            plsc.subcore_barrier()

            def body(i, carry):
                dl = []
                for b in range(2):
                    base = base0 + (2 * i + b) * CH
                    pltpu.sync_copy(dst_h.at[pl.ds(base, CH)], idx_b[b])
                    dl.append(pltpu.async_copy(g_refs[g].at[pl.ds(base, CH)],
                                               rows_b[b], lsem_b[b]))
                ds_ = []
                for b in range(2):
                    dl[b].wait()
                    ds_.append(pltpu.async_copy(rows_b[b], acc_sh.at[idx_b[b]],
                                                ssem_b[b], add=True))
                for b in range(2):
                    ds_[b].wait()
                return carry

            lax.fori_loop(0, nch // 2, body, 0)
            plsc.subcore_barrier()

            for z in range(nz):
                rb = s * rpt + z * CH
                pltpu.sync_copy(acc_sh.at[pl.ds(rb, CH)], out_refs[g].at[c, pl.ds(rb, CH)])

    return scat(*groups, dst)


@functools.partial(jax.jit, static_argnames=("W", "ng_s"))
def _acc(parts, den, bias, Sd, Pn, *, W, ng_s):
    BA = 512

    def body(*refs):
        p_refs = refs[0:ng_s]
        den_ref, bias_ref, sd_ref, pn_ref, y_ref, stats_ref, acc = refs[ng_s:]
        pid = pl.program_id(0)

        @pl.when(pid == 0)
        def _():
            acc[...] = jnp.zeros_like(acc)

        if ng_s == 1:
            p = jnp.dot(p_refs[0][0] + p_refs[0][1], pn_ref[0],
                        preferred_element_type=f32,
                        precision=lax.Precision.HIGHEST)
        else:
            p = jnp.concatenate([r[0] + r[1] for r in p_refs], axis=1)
        dsum = den_ref[0] + den_ref[1]
        dexp = jnp.dot(dsum, sd_ref[...], preferred_element_type=f32, precision=lax.Precision.HIGHEST)
        y = p / (dexp + 1e-16) + bias_ref[...]
        y_ref[...] = y
        ids = lax.broadcasted_iota(jnp.int32, (BA, 1), 0) + pid * BA
        rmask = (ids < N).astype(f32)
        ym = y * rmask
        acc[0:1, :] += jnp.sum(ym, axis=0, keepdims=True)
        acc[1:2, :] += jnp.sum(ym * y, axis=0, keepdims=True)

        @pl.when(pid == Np // BA - 1)
        def _():
            stats_ref[...] = acc[...]

    return pl.pallas_call(
        body,
        grid=(Np // BA,),
        in_specs=[pl.BlockSpec((2, BA, 128), lambda i: (0, i, 0)) for _ in range(ng_s)]
                 + [pl.BlockSpec((2, BA, 128), lambda i: (0, i, 0)),
                    pl.BlockSpec((1, W), lambda i: (0, 0)),
                    pl.BlockSpec((128, W), lambda i: (0, 0)),
                    pl.BlockSpec((1, 128, W), lambda i: (0, 0, 0))],
        out_specs=[pl.BlockSpec((BA, W), lambda i: (i, 0)),
                   pl.BlockSpec((8, W), lambda i: (0, 0))],
        out_shape=[SDS((Np, W), f32), SDS((8, W), f32)],
        scratch_shapes=[pltpu.VMEM((8, W), f32)],
    )(*parts, den, bias, Sd, Pn)


@jax.jit
def _norm(y, stats, ms, w, b):
    BN = 1024
    W = y.shape[1]

    def body(y_ref, st_ref, ms_ref, w_ref, b_ref, o_ref):
        s = st_ref[0:1, :] * (1.0 / N)
        sq = st_ref[1:2, :] * (1.0 / N)
        mm = s * ms_ref[...]
        var = sq - 2.0 * mm * s + mm * mm
        inv = lax.rsqrt(var + 1e-5)
        r = (y_ref[...] - mm) * inv * w_ref[...] + b_ref[...]
        o_ref[...] = jnp.maximum(r, 0.0) + 0.01 * jnp.minimum(r, 0.0)

    return pl.pallas_call(
        body,
        grid=(Np // BN,),
        in_specs=[pl.BlockSpec((BN, W), lambda i: (i, 0)),
                  pl.BlockSpec((8, W), lambda i: (0, 0)),
                  pl.BlockSpec((1, W), lambda i: (0, 0)),
                  pl.BlockSpec((1, W), lambda i: (0, 0)),
                  pl.BlockSpec((1, W), lambda i: (0, 0))],
        out_specs=pl.BlockSpec((BN, W), lambda i: (i, 0)),
        out_shape=SDS((Np, W), f32),
    )(y, stats, ms, w, b)


@jax.jit
def _gmp(x3, batch3d):
    BB = 512
    NB = Np // BB

    def body(x_ref, b_ref, xs_ref, accv, accc):
        pid = pl.program_id(0)

        @pl.when(pid == 0)
        def _():
            accv[...] = jnp.zeros_like(accv)
            accc[...] = jnp.zeros_like(accc)

        bvals = b_ref[0]
        oh = jnp.equal(lax.broadcasted_iota(jnp.int32, (G, BB), 0), bvals).astype(f32)
        accv[...] += jnp.dot(oh, x_ref[...], preferred_element_type=f32, precision=lax.Precision.HIGHEST)
        accc[...] += jnp.dot(oh, jnp.ones((BB, 16), f32), preferred_element_type=f32, precision=lax.Precision.HIGHEST)

        @pl.when(pid == NB - 1)
        def _():
            xs_ref[...] = accv[...] / jnp.maximum(accc[...], 1.0)

    return pl.pallas_call(
        body,
        grid=(NB,),
        in_specs=[pl.BlockSpec((BB, 16), lambda i: (i, 0)),
                  pl.BlockSpec((1, 1, BB), lambda i: (i, 0, 0))],
        out_specs=pl.BlockSpec((G, 16), lambda i: (0, 0)),
        out_shape=SDS((G, 16), f32),
        scratch_shapes=[pltpu.VMEM((G, 16), f32), pltpu.VMEM((G, 16), f32)],
    )(x3, batch3d)


@jax.jit
def _mlp(xs, W1, b1, w1n, b1n, W2, b2, w2n, b2n, W3, b3):
    def body(xs_ref, W1r, b1r, w1nr, b1nr, W2r, b2r, w2nr, b2nr, W3r, b3r,
             out_ref, sig_ref):
        def bn(h, wv, bv):
            m = jnp.mean(h, axis=0, keepdims=True)
            v = jnp.mean((h - m) * (h - m), axis=0, keepdims=True)
            return (h - m) * lax.rsqrt(v + 1e-5) * wv + bv

        lr = lambda z: jnp.maximum(z, 0.0) + 0.01 * jnp.minimum(z, 0.0)
        h = lr(bn(jnp.dot(xs_ref[...], W1r[...], preferred_element_type=f32)
                  + b1r[...], w1nr[...], b1nr[...]))
        h = lr(bn(jnp.dot(h, W2r[...], preferred_element_type=f32)
                  + b2r[...], w2nr[...], b2nr[...]))
        out = jnp.dot(h, W3r[...], preferred_element_type=f32) + b3r[...]
        out_ref[...] = out
        sig_ref[...] = jax.nn.sigmoid(out)

    return pl.pallas_call(
        body,
        out_shape=[SDS((G, 1408), f32), SDS((G, 1408), f32)],
    )(xs, W1, b1, w1n, b1n, W2, b2, w2n, b2n, W3, b3)


def _layer(x, src, dst, ea, p, gp, Hh, C):
    W = Hh * C
    fold = (W < 128)
    Wp = max(W, 128)
    ng_s = Wp // 128 if not fold else 1
    din = x.shape[1]

    S = np.zeros((Wp, H16), np.float32)
    for h in range(Hh):
        S[h * C:(h + 1) * C, h] = 1.0
    hmask = np.zeros((1, H16), np.float32)
    hmask[0, :Hh] = 1.0
    P1 = np.zeros((Wp, 128), np.float32)
    P2 = np.zeros((H16, 128), np.float32)
    if fold:
        P1[:W, :W] = np.eye(W, dtype=np.float32)
        for h in range(Hh):
            P2[h, W + h] = 1.0
        Sd = np.zeros((128, W), np.float32)
        for h in range(Hh):
            Sd[W + h, h * C:(h + 1) * C] = 1.0
    else:
        P2[:H16, :H16] = np.eye(H16, dtype=np.float32)
        Sd = np.zeros((128, W), np.float32)
        for h in range(Hh):
            Sd[h, h * C:(h + 1) * C] = 1.0
    Pn = np.zeros((1, 128, W), np.float32)
    Pn[0, :min(128, W), :min(128, W)] = np.eye(min(128, W), dtype=np.float32)

    padW = ((0, 0), (0, Wp - W))
    Wl = jnp.pad(p['Wl'], padW)
    Wr = jnp.pad(p['Wr'], padW)
    bl = jnp.pad(p['bl'], (0, Wp - W)).reshape(1, Wp)
    br = jnp.pad(p['br'], (0, Wp - W)).reshape(1, Wp)
    attb = jnp.pad(p['att'].reshape(1, W), ((0, 0), (0, Wp - W)))
    We = jnp.pad(p['We'], ((0, H16 - p['We'].shape[0]), (0, Wp - W)))

    xl, xr = _lin2(x, Wl, bl, Wr, br, din=din, W=Wp)
    xls, xrd = _sc_gather(xl, xr, src, dst, W=Wp)
    groups = _att(xls, xrd, ea, We, attb, jnp.asarray(S), jnp.asarray(S.T),
                  jnp.asarray(hmask), jnp.asarray(P1), jnp.asarray(P2),
                  W=Wp, ng_s=ng_s, fold=fold)
    gparts = _sc_scatter(list(groups), dst, ng=len(groups))
    sparts = list(gparts[:ng_s]) if not fold else [gparts[0]]
    dpart = gparts[-1]
    y, stats = _acc(sparts, dpart, p['bias'].reshape(1, W), jnp.asarray(Sd),
                    jnp.asarray(Pn), W=W, ng_s=ng_s)
    return _norm(y, stats, gp['ms'].reshape(1, W), gp['w'].reshape(1, W),
                 gp['b'].reshape(1, W))


def kernel(x_s, edge_attr_s, x_t, edge_attr_t, params, edge_index_s,
           edge_index_t, xs_batch, xt_batch):
    p = params
    x = jnp.pad(x_s, ((0, Np - N), (0, 0)))
    src = jnp.concatenate([edge_index_s[0], jnp.full((Ep - E,), N, jnp.int32)])
    dst = jnp.concatenate([edge_index_s[1], jnp.full((Ep - E,), N, jnp.int32)])
    ea = jnp.pad(edge_attr_s, ((0, Ep - E), (0, H16 - 9)))
    batch3d = jnp.concatenate([xs_batch, jnp.full((Np - N,), G, jnp.int32)]
                              ).reshape(Np // 512, 1, 512)

    x1 = _layer(x, src, dst, ea, p['g1'], p['gn1'], 8, 64)
    x2 = _layer(x1, src, dst, ea, p['g2'], p['gn2'], 4, 32)
    x3 = _layer(x2, src, dst, ea, p['g3'], p['gn3'], 1, 16)
    xs = _gmp(x3, batch3d)

    def pad_lin(lp, ko, no):
        return (jnp.pad(lp['W'], ((0, ko - lp['W'].shape[0]), (0, no - lp['W'].shape[1]))),
                jnp.pad(lp['b'], (0, no - lp['b'].shape[0])).reshape(1, no))

    W1, b1 = pad_lin(p['lin1'], 16, 384)
    W2, b2 = pad_lin(p['lin2'], 384, 768)
    W3, b3 = pad_lin(p['lin3'], 768, 1408)
    w1n = jnp.pad(p['bn1']['w'], (0, 384 - 329)).reshape(1, 384)
    b1n = jnp.pad(p['bn1']['b'], (0, 384 - 329)).reshape(1, 384)
    w2n = jnp.pad(p['bn2']['w'], (0, 768 - 658)).reshape(1, 768)
    b2n = jnp.pad(p['bn2']['b'], (0, 768 - 658)).reshape(1, 768)
    out_p, sig_p = _mlp(xs, W1, b1, w1n, b1n, W2, b2, w2n, b2n, W3, b3)
    return (out_p[:, :1317], sig_p[:, :1317])


# speedup vs baseline: 8.8318x; 1.0249x over previous
import functools

import jax
import jax.numpy as jnp
import numpy as np
from jax import lax
from jax.experimental import pallas as pl
from jax.experimental.pallas import tpu as pltpu
from jax.experimental.pallas import tpu_sc as plsc

N = 10000
E = 160000
D = 128
G = 64
Np = 10240
Ep = 163840
H16 = 16
CH = 128
NW = 32

SDS = jax.ShapeDtypeStruct
f32 = jnp.float32


def _mesh():
    return plsc.VectorSubcoreMesh(core_axis_name="c", subcore_axis_name="s")


@functools.partial(jax.jit, static_argnames=("din", "W"))
def _lin2(x, Wl, bl, Wr, br, *, din, W):
    BN = 512

    def body(x_ref, wl_ref, bl_ref, wr_ref, br_ref, xl_ref, xr_ref):
        xb = x_ref[...]
        xl_ref[...] = jnp.dot(xb, wl_ref[...], preferred_element_type=f32) + bl_ref[...]
        xr_ref[...] = jnp.dot(xb, wr_ref[...], preferred_element_type=f32) + br_ref[...]

    return pl.pallas_call(
        body,
        grid=(Np // BN,),
        in_specs=[
            pl.BlockSpec((BN, din), lambda i: (i, 0)),
            pl.BlockSpec((din, W), lambda i: (0, 0)),
            pl.BlockSpec((1, W), lambda i: (0, 0)),
            pl.BlockSpec((din, W), lambda i: (0, 0)),
            pl.BlockSpec((1, W), lambda i: (0, 0)),
        ],
        out_specs=[pl.BlockSpec((BN, W), lambda i: (i, 0)),
                   pl.BlockSpec((BN, W), lambda i: (i, 0))],
        out_shape=[SDS((Np, W), f32), SDS((Np, W), f32)],
    )(x, Wl, bl, Wr, br)


@functools.partial(jax.jit, static_argnames=("W",))
def _sc_gather(xl, xr, src, dst, *, W):
    ch = 80 if W > 128 else CH
    nbuf = 2 if W > 128 else 4
    nper = Ep // NW
    nch = nper // ch
    src2 = src.reshape(Ep // ch, ch)
    dst2 = dst.reshape(Ep // ch, ch)

    @functools.partial(
        pl.kernel,
        mesh=_mesh(),
        out_type=(SDS((Ep, W), f32), SDS((Ep, W), f32)),
        scratch_types=(pltpu.VMEM((nch, ch), jnp.int32),
                       pltpu.VMEM((nch, ch), jnp.int32),
                       tuple(pltpu.VMEM((ch, W), f32) for _ in range(nbuf)),
                       tuple(pltpu.SemaphoreType.DMA for _ in range(nbuf)),
    ))
    def gat(xl_h, xr_h, src_h, dst_h, xls_o, xrd_o, sidx, didx, rows, sems):
        wid = lax.axis_index("s") * 2 + lax.axis_index("c")
        base0 = wid * nper
        row0 = wid * nch
        pltpu.sync_copy(src_h.at[pl.ds(row0, nch)], sidx)
        pltpu.sync_copy(dst_h.at[pl.ds(row0, nch)], didx)
        npair = nbuf // 2

        def body(i, carry):
            dl = []
            for b in range(npair):
                k = npair * i + b
                dl.append(pltpu.async_copy(xl_h.at[sidx.at[k]], rows[2 * b], sems[2 * b]))
                dl.append(pltpu.async_copy(xr_h.at[didx.at[k]], rows[2 * b + 1], sems[2 * b + 1]))
            for b in range(npair):
                k = npair * i + b
                base = base0 + k * ch
                dl[2 * b].wait()
                pltpu.sync_copy(rows[2 * b], xls_o.at[pl.ds(base, ch)])
                dl[2 * b + 1].wait()
                pltpu.sync_copy(rows[2 * b + 1], xrd_o.at[pl.ds(base, ch)])
            return carry

        lax.fori_loop(0, nch // npair, body, 0)

    return gat(xl, xr, src2, dst2)


@functools.partial(jax.jit, static_argnames=("W", "ng_s", "fold"))
def _att(xls, xrd, ea, We, attb, S, St, hmask, P1, P2, *, W, ng_s, fold):
    BE = 512
    ng_out = ng_s if fold else ng_s + 1

    def body(xls_ref, xrd_ref, ea_ref, we_ref, attb_ref, s_ref, st_ref, hm_ref,
             p1_ref, p2_ref, *outs):
        pid = pl.program_id(0)
        xlsb = xls_ref[...]
        m = xlsb + xrd_ref[...] + jnp.dot(ea_ref[...], we_ref[...],
                                          preferred_element_type=f32)
        m = jnp.maximum(m, 0.0) + 0.2 * jnp.minimum(m, 0.0)
        logits = jnp.dot(m * attb_ref[...], s_ref[...], preferred_element_type=f32, precision=lax.Precision.HIGHEST)
        ids = lax.broadcasted_iota(jnp.int32, (BE, 1), 0) + pid * BE
        emask = (ids < E).astype(f32)
        ex = jnp.exp(logits) * hm_ref[...] * emask
        scaled = xlsb * jnp.dot(ex, st_ref[...], preferred_element_type=f32, precision=lax.Precision.HIGHEST)
        exg = jnp.dot(ex, p2_ref[...], preferred_element_type=f32, precision=lax.Precision.HIGHEST)
        if fold:
            outs[0][...] = jnp.dot(scaled, p1_ref[...], preferred_element_type=f32, precision=lax.Precision.HIGHEST) + exg
        else:
            for g in range(ng_s):
                outs[g][...] = scaled[:, g * 128:(g + 1) * 128]
            outs[ng_s][...] = exg

    return pl.pallas_call(
        body,
        grid=(Ep // BE,),
        in_specs=[
            pl.BlockSpec((BE, W), lambda i: (i, 0)),
            pl.BlockSpec((BE, W), lambda i: (i, 0)),
            pl.BlockSpec((BE, H16), lambda i: (i, 0)),
            pl.BlockSpec((H16, W), lambda i: (0, 0)),
            pl.BlockSpec((1, W), lambda i: (0, 0)),
            pl.BlockSpec((W, H16), lambda i: (0, 0)),
            pl.BlockSpec((H16, W), lambda i: (0, 0)),
            pl.BlockSpec((1, H16), lambda i: (0, 0)),
            pl.BlockSpec((W, 128), lambda i: (0, 0)),
            pl.BlockSpec((H16, 128), lambda i: (0, 0)),
        ],
        out_specs=[pl.BlockSpec((BE, 128), lambda i: (i, 0)) for _ in range(ng_out)],
        out_shape=[SDS((Ep, 128), f32) for _ in range(ng_out)],
    )(xls, xrd, ea, We, attb, S, St, hmask, P1, P2)


@functools.partial(jax.jit, static_argnames=("ng",))
def _sc_scatter(groups, dst, *, ng):
    nper = Ep // NW
    nch = nper // CH
    rpt = Np // 16
    nz = rpt // CH

    @functools.partial(
        pl.kernel,
        mesh=_mesh(),
        out_type=tuple(SDS((2, Np, 128), f32) for _ in range(ng)),
        scratch_types=(pltpu.VMEM((Ep // NW // CH, CH), jnp.int32),
                       pltpu.VMEM((CH, 128), f32),
                       pltpu.VMEM((CH, 128), f32),
                       pltpu.VMEM_SHARED((Np, 128), f32),
                       pltpu.SemaphoreType.DMA,
                       pltpu.SemaphoreType.DMA,
                       pltpu.SemaphoreType.DMA,
                       pltpu.SemaphoreType.DMA),
    )
    def scat(*refs):
        g_refs = refs[0:ng]
        dst_h = refs[ng]
        out_refs = refs[ng + 1:2 * ng + 1]
        didx, rows0, rows1, acc_sh, lsem0, lsem1, ssem0, ssem1 = refs[2 * ng + 1:]
        rows_b = (rows0, rows1)
        lsem_b = (lsem0, lsem1)
        ssem_b = (ssem0, ssem1)
        rows_v = rows0

        c = lax.axis_index("c")
        s = lax.axis_index("s")
        wid = s * 2 + c
        base0 = wid * nper
        pltpu.sync_copy(dst_h.at[pl.ds(wid * nch, nch)], didx)
        zero16 = jnp.zeros((16,), f32)

        for g in range(ng):
            def zrow(i, carry):
                for j in range(8):
                    rows_v[i, pl.ds(j * 16, 16)] = zero16
                return carry
            lax.fori_loop(0, CH, zrow, 0)
            for z in range(nz):
                pltpu.sync_copy(rows_v, acc_sh.at[pl.ds(s * rpt + z * CH, CH)])
            plsc.subcore_barrier()

            def body(i, carry):
                dl = []
                for b in range(2):
                    base = base0 + (2 * i + b) * CH
                    dl.append(pltpu.async_copy(g_refs[g].at[pl.ds(base, CH)],
                                               rows_b[b], lsem_b[b]))
                ds_ = []
                for b in range(2):
                    dl[b].wait()
                    ds_.append(pltpu.async_copy(rows_b[b],
                                                acc_sh.at[didx.at[2 * i + b]],
                                                ssem_b[b], add=True))
                for b in range(2):
                    ds_[b].wait()
                return carry

            lax.fori_loop(0, nch // 2, body, 0)
            plsc.subcore_barrier()

            for z in range(nz):
                rb = s * rpt + z * CH
                pltpu.sync_copy(acc_sh.at[pl.ds(rb, CH)], out_refs[g].at[c, pl.ds(rb, CH)])

    return scat(*groups, dst.reshape(Ep // CH, CH))


@functools.partial(jax.jit, static_argnames=("W", "ng_s"))
def _acc(parts, den, bias, Sd, Pn, *, W, ng_s):
    BA = 512

    def body(*refs):
        p_refs = refs[0:ng_s]
        den_ref, bias_ref, sd_ref, pn_ref, y_ref, stats_ref, acc = refs[ng_s:]
        pid = pl.program_id(0)

        @pl.when(pid == 0)
        def _():
            acc[...] = jnp.zeros_like(acc)

        if ng_s == 1:
            p = jnp.dot(p_refs[0][0] + p_refs[0][1], pn_ref[0],
                        preferred_element_type=f32,
                        precision=lax.Precision.HIGHEST)
        else:
            p = jnp.concatenate([r[0] + r[1] for r in p_refs], axis=1)
        dsum = den_ref[0] + den_ref[1]
        dexp = jnp.dot(dsum, sd_ref[...], preferred_element_type=f32, precision=lax.Precision.HIGHEST)
        y = p / (dexp + 1e-16) + bias_ref[...]
        y_ref[...] = y
        ids = lax.broadcasted_iota(jnp.int32, (BA, 1), 0) + pid * BA
        rmask = (ids < N).astype(f32)
        ym = y * rmask
        acc[0:1, :] += jnp.sum(ym, axis=0, keepdims=True)
        acc[1:2, :] += jnp.sum(ym * y, axis=0, keepdims=True)

        @pl.when(pid == Np // BA - 1)
        def _():
            stats_ref[...] = acc[...]

    return pl.pallas_call(
        body,
        grid=(Np // BA,),
        in_specs=[pl.BlockSpec((2, BA, 128), lambda i: (0, i, 0)) for _ in range(ng_s)]
                 + [pl.BlockSpec((2, BA, 128), lambda i: (0, i, 0)),
                    pl.BlockSpec((1, W), lambda i: (0, 0)),
                    pl.BlockSpec((128, W), lambda i: (0, 0)),
                    pl.BlockSpec((1, 128, W), lambda i: (0, 0, 0))],
        out_specs=[pl.BlockSpec((BA, W), lambda i: (i, 0)),
                   pl.BlockSpec((8, W), lambda i: (0, 0))],
        out_shape=[SDS((Np, W), f32), SDS((8, W), f32)],
        scratch_shapes=[pltpu.VMEM((8, W), f32)],
    )(*parts, den, bias, Sd, Pn)


@jax.jit
def _norm(y, stats, ms, w, b):
    BN = 1024
    W = y.shape[1]

    def body(y_ref, st_ref, ms_ref, w_ref, b_ref, o_ref):
        s = st_ref[0:1, :] * (1.0 / N)
        sq = st_ref[1:2, :] * (1.0 / N)
        mm = s * ms_ref[...]
        var = sq - 2.0 * mm * s + mm * mm
        inv = lax.rsqrt(var + 1e-5)
        r = (y_ref[...] - mm) * inv * w_ref[...] + b_ref[...]
        o_ref[...] = jnp.maximum(r, 0.0) + 0.01 * jnp.minimum(r, 0.0)

    return pl.pallas_call(
        body,
        grid=(Np // BN,),
        in_specs=[pl.BlockSpec((BN, W), lambda i: (i, 0)),
                  pl.BlockSpec((8, W), lambda i: (0, 0)),
                  pl.BlockSpec((1, W), lambda i: (0, 0)),
                  pl.BlockSpec((1, W), lambda i: (0, 0)),
                  pl.BlockSpec((1, W), lambda i: (0, 0))],
        out_specs=pl.BlockSpec((BN, W), lambda i: (i, 0)),
        out_shape=SDS((Np, W), f32),
    )(y, stats, ms, w, b)


@jax.jit
def _gmp(x3, batch3d):
    BB = 512
    NB = Np // BB

    def body(x_ref, b_ref, xs_ref, accv, accc):
        pid = pl.program_id(0)

        @pl.when(pid == 0)
        def _():
            accv[...] = jnp.zeros_like(accv)
            accc[...] = jnp.zeros_like(accc)

        bvals = b_ref[0]
        oh = jnp.equal(lax.broadcasted_iota(jnp.int32, (G, BB), 0), bvals).astype(f32)
        accv[...] += jnp.dot(oh, x_ref[...], preferred_element_type=f32, precision=lax.Precision.HIGHEST)
        accc[...] += jnp.dot(oh, jnp.ones((BB, 16), f32), preferred_element_type=f32, precision=lax.Precision.HIGHEST)

        @pl.when(pid == NB - 1)
        def _():
            xs_ref[...] = accv[...] / jnp.maximum(accc[...], 1.0)

    return pl.pallas_call(
        body,
        grid=(NB,),
        in_specs=[pl.BlockSpec((BB, 16), lambda i: (i, 0)),
                  pl.BlockSpec((1, 1, BB), lambda i: (i, 0, 0))],
        out_specs=pl.BlockSpec((G, 16), lambda i: (0, 0)),
        out_shape=SDS((G, 16), f32),
        scratch_shapes=[pltpu.VMEM((G, 16), f32), pltpu.VMEM((G, 16), f32)],
    )(x3, batch3d)


@jax.jit
def _mlp(xs, W1, b1, w1n, b1n, W2, b2, w2n, b2n, W3, b3):
    def body(xs_ref, W1r, b1r, w1nr, b1nr, W2r, b2r, w2nr, b2nr, W3r, b3r,
             out_ref, sig_ref):
        def bn(h, wv, bv):
            m = jnp.mean(h, axis=0, keepdims=True)
            v = jnp.mean((h - m) * (h - m), axis=0, keepdims=True)
            return (h - m) * lax.rsqrt(v + 1e-5) * wv + bv

        lr = lambda z: jnp.maximum(z, 0.0) + 0.01 * jnp.minimum(z, 0.0)
        h = lr(bn(jnp.dot(xs_ref[...], W1r[...], preferred_element_type=f32)
                  + b1r[...], w1nr[...], b1nr[...]))
        h = lr(bn(jnp.dot(h, W2r[...], preferred_element_type=f32)
                  + b2r[...], w2nr[...], b2nr[...]))
        out = jnp.dot(h, W3r[...], preferred_element_type=f32) + b3r[...]
        out_ref[...] = out
        sig_ref[...] = jax.nn.sigmoid(out)

    return pl.pallas_call(
        body,
        out_shape=[SDS((G, 1408), f32), SDS((G, 1408), f32)],
    )(xs, W1, b1, w1n, b1n, W2, b2, w2n, b2n, W3, b3)


def _layer(x, src, dst, ea, p, gp, Hh, C):
    W = Hh * C
    fold = (W < 128)
    Wp = max(W, 128)
    ng_s = Wp // 128 if not fold else 1
    din = x.shape[1]

    S = np.zeros((Wp, H16), np.float32)
    for h in range(Hh):
        S[h * C:(h + 1) * C, h] = 1.0
    hmask = np.zeros((1, H16), np.float32)
    hmask[0, :Hh] = 1.0
    P1 = np.zeros((Wp, 128), np.float32)
    P2 = np.zeros((H16, 128), np.float32)
    if fold:
        P1[:W, :W] = np.eye(W, dtype=np.float32)
        for h in range(Hh):
            P2[h, W + h] = 1.0
        Sd = np.zeros((128, W), np.float32)
        for h in range(Hh):
            Sd[W + h, h * C:(h + 1) * C] = 1.0
    else:
        P2[:H16, :H16] = np.eye(H16, dtype=np.float32)
        Sd = np.zeros((128, W), np.float32)
        for h in range(Hh):
            Sd[h, h * C:(h + 1) * C] = 1.0
    Pn = np.zeros((1, 128, W), np.float32)
    Pn[0, :min(128, W), :min(128, W)] = np.eye(min(128, W), dtype=np.float32)

    padW = ((0, 0), (0, Wp - W))
    Wl = jnp.pad(p['Wl'], padW)
    Wr = jnp.pad(p['Wr'], padW)
    bl = jnp.pad(p['bl'], (0, Wp - W)).reshape(1, Wp)
    br = jnp.pad(p['br'], (0, Wp - W)).reshape(1, Wp)
    attb = jnp.pad(p['att'].reshape(1, W), ((0, 0), (0, Wp - W)))
    We = jnp.pad(p['We'], ((0, H16 - p['We'].shape[0]), (0, Wp - W)))

    xl, xr = _lin2(x, Wl, bl, Wr, br, din=din, W=Wp)
    xls, xrd = _sc_gather(xl, xr, src, dst, W=Wp)
    groups = _att(xls, xrd, ea, We, attb, jnp.asarray(S), jnp.asarray(S.T),
                  jnp.asarray(hmask), jnp.asarray(P1), jnp.asarray(P2),
                  W=Wp, ng_s=ng_s, fold=fold)
    gparts = _sc_scatter(list(groups), dst, ng=len(groups))
    sparts = list(gparts[:ng_s]) if not fold else [gparts[0]]
    dpart = gparts[-1]
    y, stats = _acc(sparts, dpart, p['bias'].reshape(1, W), jnp.asarray(Sd),
                    jnp.asarray(Pn), W=W, ng_s=ng_s)
    return _norm(y, stats, gp['ms'].reshape(1, W), gp['w'].reshape(1, W),
                 gp['b'].reshape(1, W))


def kernel(x_s, edge_attr_s, x_t, edge_attr_t, params, edge_index_s,
           edge_index_t, xs_batch, xt_batch):
    p = params
    x = jnp.pad(x_s, ((0, Np - N), (0, 0)))
    src = jnp.concatenate([edge_index_s[0], jnp.full((Ep - E,), N, jnp.int32)])
    dst = jnp.concatenate([edge_index_s[1], jnp.full((Ep - E,), N, jnp.int32)])
    ea = jnp.pad(edge_attr_s, ((0, Ep - E), (0, H16 - 9)))
    batch3d = jnp.concatenate([xs_batch, jnp.full((Np - N,), G, jnp.int32)]
                              ).reshape(Np // 512, 1, 512)

    x1 = _layer(x, src, dst, ea, p['g1'], p['gn1'], 8, 64)
    x2 = _layer(x1, src, dst, ea, p['g2'], p['gn2'], 4, 32)
    x3 = _layer(x2, src, dst, ea, p['g3'], p['gn3'], 1, 16)
    xs = _gmp(x3, batch3d)

    def pad_lin(lp, ko, no):
        return (jnp.pad(lp['W'], ((0, ko - lp['W'].shape[0]), (0, no - lp['W'].shape[1]))),
                jnp.pad(lp['b'], (0, no - lp['b'].shape[0])).reshape(1, no))

    W1, b1 = pad_lin(p['lin1'], 16, 384)
    W2, b2 = pad_lin(p['lin2'], 384, 768)
    W3, b3 = pad_lin(p['lin3'], 768, 1408)
    w1n = jnp.pad(p['bn1']['w'], (0, 384 - 329)).reshape(1, 384)
    b1n = jnp.pad(p['bn1']['b'], (0, 384 - 329)).reshape(1, 384)
    w2n = jnp.pad(p['bn2']['w'], (0, 768 - 658)).reshape(1, 768)
    b2n = jnp.pad(p['bn2']['b'], (0, 768 - 658)).reshape(1, 768)
    out_p, sig_p = _mlp(xs, W1, b1, w1n, b1n, W2, b2, w2n, b2n, W3, b3)
    return (out_p[:, :1317], sig_p[:, :1317])
